# Initial kernel scaffold; baseline (speedup 1.0000x reference)
import jax
import jax.numpy as jnp
from jax.experimental import pallas as pl


def kernel(x, edge_index_1, edge_index_2, W1_1, W1_2, W2_1, W2_2):
    raise NotImplementedError("write your pallas kernel here")


# speedup vs baseline: 3.5491x; 3.5491x over previous
import functools

import jax
import jax.numpy as jnp
from jax import lax
from jax.experimental import pallas as pl
from jax.experimental.pallas import tpu as pltpu
from jax.experimental.pallas import tpu_sc as plsc

N = 10000
E = 160000
D_IN = 128
H = 64

NC = 2
NS = 16
NW = NC * NS
CH = 128
E_PAD = 163840
NCH1 = E_PAD // (NW * CH)
NCH = 2 * NCH1
NPAD = 10240
ACC_ROWS = 2 * NPAD
ZCH = ACC_ROWS // NS // CH
TROWS = 2 * N

BLK = 400


def _prep_edges(edge_index_1, edge_index_2):
    pad = E_PAD - E
    zpad = jnp.zeros((pad,), jnp.int32)
    src1 = jnp.concatenate([edge_index_1[0], zpad])
    dst1 = jnp.concatenate([edge_index_1[1], jnp.full((pad,), N, jnp.int32)])
    src2 = jnp.concatenate([edge_index_2[0] + N, zpad])
    dst2 = jnp.concatenate(
        [edge_index_2[1] + NPAD, jnp.full((pad,), NPAD + N, jnp.int32)])
    s1 = src1.reshape(NW, NCH1, CH)
    s2 = src2.reshape(NW, NCH1, CH)
    d1 = dst1.reshape(NW, NCH1, CH)
    d2 = dst2.reshape(NW, NCH1, CH)
    src_idx = jnp.concatenate([s1, s2], axis=1)
    dst_idx = jnp.concatenate([d1, d2], axis=1)
    return src_idx, dst_idx


def _sc_propagate(table, src_idx, dst_idx):
    mesh = plsc.VectorSubcoreMesh(core_axis_name="c", subcore_axis_name="s")

    @functools.partial(
        pl.kernel,
        out_type=jax.ShapeDtypeStruct((NC * ACC_ROWS, H), jnp.float32),
        mesh=mesh,
        scratch_types=[
            pltpu.VMEM((NCH, CH), jnp.int32),
            pltpu.VMEM((NCH, CH), jnp.int32),
            pltpu.VMEM((CH, H), jnp.float32),
            pltpu.VMEM_SHARED((ACC_ROWS, H), jnp.float32),
            pltpu.SemaphoreType.DMA,
        ],
        compiler_params=pltpu.CompilerParams(use_tc_tiling_on_sc=False),
    )
    def prop(table_hbm, src_hbm, dst_hbm, out_hbm, src_v, dst_v, rows_v,
             acc_sh, sem):
        c = lax.axis_index("c")
        s = lax.axis_index("s")
        w = c * NS + s

        def zrow(r, carry):
            for k in range(H // 16):
                rows_v[r, pl.ds(k * 16, 16)] = jnp.zeros((16,), jnp.float32)
            return carry
        lax.fori_loop(0, CH, zrow, 0)
        zbase = s * (ACC_ROWS // NS)

        def zcp(i, carry):
            pltpu.sync_copy(rows_v, acc_sh.at[pl.ds(zbase + i * CH, CH)])
            return carry
        lax.fori_loop(0, ZCH, zcp, 0)

        pltpu.sync_copy(src_hbm.at[w], src_v)
        pltpu.sync_copy(dst_hbm.at[w], dst_v)
        plsc.subcore_barrier()

        def step(j, carry):
            pltpu.async_copy(table_hbm.at[src_v.at[j]], rows_v, sem).wait()
            pltpu.sync_copy(rows_v, acc_sh.at[dst_v.at[j]], add=True)
            return carry
        lax.fori_loop(0, NCH, step, 0)
        plsc.subcore_barrier()

        pltpu.sync_copy(
            acc_sh.at[pl.ds(zbase, ACC_ROWS // NS)],
            out_hbm.at[pl.ds(c * ACC_ROWS + zbase, ACC_ROWS // NS)])

    return prop(table, src_idx, dst_idx)


def _tc_project_l1(x, w_stack):
    def body(x_ref, w_ref, o_ref):
        o_ref[0] = jnp.dot(x_ref[...], w_ref[0],
                           preferred_element_type=jnp.float32)

    out = pl.pallas_call(
        body,
        grid=(N // BLK, 2),
        in_specs=[
            pl.BlockSpec((BLK, D_IN), lambda i, j: (i, 0)),
            pl.BlockSpec((1, D_IN, H), lambda i, j: (j, 0, 0)),
        ],
        out_specs=pl.BlockSpec((1, BLK, H), lambda i, j: (j, i, 0)),
        out_shape=jax.ShapeDtypeStruct((2, N, H), jnp.float32),
    )(x, w_stack)
    return out.reshape(TROWS, H)


def _tc_combine_project(p00, p01, p10, p11, w_stack):
    def body(a_ref, b_ref, c_ref, d_ref, w_ref, o_ref):
        h = jnp.maximum(
            a_ref[...] + b_ref[...] + c_ref[...] + d_ref[...], 0.0)
        o_ref[0] = jnp.dot(h, w_ref[0], preferred_element_type=jnp.float32)

    part_spec = pl.BlockSpec((BLK, H), lambda i, j: (i, 0))
    out = pl.pallas_call(
        body,
        grid=(N // BLK, 2),
        in_specs=[part_spec, part_spec, part_spec, part_spec,
                  pl.BlockSpec((1, H, H), lambda i, j: (j, 0, 0))],
        out_specs=pl.BlockSpec((1, BLK, H), lambda i, j: (j, i, 0)),
        out_shape=jax.ShapeDtypeStruct((2, N, H), jnp.float32),
    )(p00, p01, p10, p11, w_stack)
    return out.reshape(TROWS, H)


def _tc_combine(p00, p01, p10, p11):
    def body(a_ref, b_ref, c_ref, d_ref, o_ref):
        o_ref[...] = jnp.maximum(
            a_ref[...] + b_ref[...] + c_ref[...] + d_ref[...], 0.0)

    part_spec = pl.BlockSpec((BLK, H), lambda i: (i, 0))
    return pl.pallas_call(
        body,
        grid=(N // BLK,),
        in_specs=[part_spec, part_spec, part_spec, part_spec],
        out_specs=part_spec,
        out_shape=jax.ShapeDtypeStruct((N, H), jnp.float32),
    )(p00, p01, p10, p11)


def _split_parts(parts):
    return (parts[0:N], parts[NPAD:NPAD + N],
            parts[2 * NPAD:2 * NPAD + N], parts[3 * NPAD:3 * NPAD + N])


def kernel(x, edge_index_1, edge_index_2, W1_1, W1_2, W2_1, W2_2):
    src_idx, dst_idx = _prep_edges(edge_index_1, edge_index_2)

    table1 = _tc_project_l1(x, jnp.stack([W1_1, W1_2]))
    parts1 = _sc_propagate(table1, src_idx, dst_idx)
    table2 = _tc_combine_project(*_split_parts(parts1),
                                 jnp.stack([W2_1, W2_2]))
    parts2 = _sc_propagate(table2, src_idx, dst_idx)
    return _tc_combine(*_split_parts(parts2))


# speedup vs baseline: 4.0308x; 1.1357x over previous
import functools

import jax
import jax.numpy as jnp
from jax import lax
from jax.experimental import pallas as pl
from jax.experimental.pallas import tpu as pltpu
from jax.experimental.pallas import tpu_sc as plsc

N = 10000
E = 160000
D_IN = 128
H = 64

NC = 2
NS = 16
NW = NC * NS
CH = 128
E_PAD = 163840
NCH1 = E_PAD // (NW * CH)
NCH = 2 * NCH1
NPAD = 10240
ACC_ROWS = 2 * NPAD
ZCH = ACC_ROWS // NS // CH
TROWS = 2 * N

BLK = 400


def _prep_edges(edge_index_1, edge_index_2):
    pad = E_PAD - E
    zpad = jnp.zeros((pad,), jnp.int32)
    src1 = jnp.concatenate([edge_index_1[0], zpad])
    dst1 = jnp.concatenate([edge_index_1[1], jnp.full((pad,), N, jnp.int32)])
    src2 = jnp.concatenate([edge_index_2[0] + N, zpad])
    dst2 = jnp.concatenate(
        [edge_index_2[1] + NPAD, jnp.full((pad,), NPAD + N, jnp.int32)])
    s1 = src1.reshape(NW, NCH1, CH)
    s2 = src2.reshape(NW, NCH1, CH)
    d1 = dst1.reshape(NW, NCH1, CH)
    d2 = dst2.reshape(NW, NCH1, CH)
    src_idx = jnp.concatenate([s1, s2], axis=1)
    dst_idx = jnp.concatenate([d1, d2], axis=1)
    return src_idx, dst_idx


def _sc_propagate(table, src_idx, dst_idx):
    mesh = plsc.VectorSubcoreMesh(core_axis_name="c", subcore_axis_name="s")

    @functools.partial(
        pl.kernel,
        out_type=jax.ShapeDtypeStruct((NC * ACC_ROWS, H), jnp.float32),
        mesh=mesh,
        scratch_types=[
            pltpu.VMEM((NCH, CH), jnp.int32),
            pltpu.VMEM((NCH, CH), jnp.int32),
            pltpu.VMEM((CH, H), jnp.float32),
            pltpu.VMEM((CH, H), jnp.float32),
            pltpu.VMEM_SHARED((ACC_ROWS, H), jnp.float32),
            pltpu.SemaphoreType.DMA,
            pltpu.SemaphoreType.DMA,
        ],
        compiler_params=pltpu.CompilerParams(use_tc_tiling_on_sc=False),
    )
    def prop(table_hbm, src_hbm, dst_hbm, out_hbm, src_v, dst_v, rows_a,
             rows_b, acc_sh, sem_a, sem_b):
        c = lax.axis_index("c")
        s = lax.axis_index("s")
        w = c * NS + s

        def zrow(r, carry):
            for k in range(H // 16):
                rows_a[r, pl.ds(k * 16, 16)] = jnp.zeros((16,), jnp.float32)
            return carry
        lax.fori_loop(0, CH, zrow, 0)
        zbase = s * (ACC_ROWS // NS)

        def zcp(i, carry):
            pltpu.sync_copy(rows_a, acc_sh.at[pl.ds(zbase + i * CH, CH)])
            return carry
        lax.fori_loop(0, ZCH, zcp, 0)

        pltpu.sync_copy(src_hbm.at[w], src_v)
        pltpu.sync_copy(dst_hbm.at[w], dst_v)
        plsc.subcore_barrier()

        bufs = ((rows_a, sem_a), (rows_b, sem_b))
        pltpu.async_copy(table_hbm.at[src_v.at[0]], rows_a, sem_a)

        def step2(i, carry):
            for k in range(2):
                j = 2 * i + k
                rows_c, sem_c = bufs[k]
                rows_n, sem_n = bufs[1 - k]

                @pl.when(j + 1 < NCH)
                def _():
                    pltpu.async_copy(
                        table_hbm.at[src_v.at[j + 1]], rows_n, sem_n)
                pltpu.make_async_copy(
                    table_hbm.at[src_v.at[j]], rows_c, sem_c).wait()
                pltpu.sync_copy(rows_c, acc_sh.at[dst_v.at[j]], add=True)
            return carry
        lax.fori_loop(0, NCH // 2, step2, 0)
        plsc.subcore_barrier()

        pltpu.sync_copy(
            acc_sh.at[pl.ds(zbase, ACC_ROWS // NS)],
            out_hbm.at[pl.ds(c * ACC_ROWS + zbase, ACC_ROWS // NS)])

    return prop(table, src_idx, dst_idx)


def _tc_project_l1(x, w_stack):
    def body(x_ref, w_ref, o_ref):
        o_ref[0] = jnp.dot(x_ref[...], w_ref[0],
                           preferred_element_type=jnp.float32)

    out = pl.pallas_call(
        body,
        grid=(N // BLK, 2),
        in_specs=[
            pl.BlockSpec((BLK, D_IN), lambda i, j: (i, 0)),
            pl.BlockSpec((1, D_IN, H), lambda i, j: (j, 0, 0)),
        ],
        out_specs=pl.BlockSpec((1, BLK, H), lambda i, j: (j, i, 0)),
        out_shape=jax.ShapeDtypeStruct((2, N, H), jnp.float32),
    )(x, w_stack)
    return out.reshape(TROWS, H)


def _tc_combine_project(p00, p01, p10, p11, w_stack):
    def body(a_ref, b_ref, c_ref, d_ref, w_ref, o_ref):
        h = jnp.maximum(
            a_ref[...] + b_ref[...] + c_ref[...] + d_ref[...], 0.0)
        o_ref[0] = jnp.dot(h, w_ref[0], preferred_element_type=jnp.float32)

    part_spec = pl.BlockSpec((BLK, H), lambda i, j: (i, 0))
    out = pl.pallas_call(
        body,
        grid=(N // BLK, 2),
        in_specs=[part_spec, part_spec, part_spec, part_spec,
                  pl.BlockSpec((1, H, H), lambda i, j: (j, 0, 0))],
        out_specs=pl.BlockSpec((1, BLK, H), lambda i, j: (j, i, 0)),
        out_shape=jax.ShapeDtypeStruct((2, N, H), jnp.float32),
    )(p00, p01, p10, p11, w_stack)
    return out.reshape(TROWS, H)


def _tc_combine(p00, p01, p10, p11):
    def body(a_ref, b_ref, c_ref, d_ref, o_ref):
        o_ref[...] = jnp.maximum(
            a_ref[...] + b_ref[...] + c_ref[...] + d_ref[...], 0.0)

    part_spec = pl.BlockSpec((BLK, H), lambda i: (i, 0))
    return pl.pallas_call(
        body,
        grid=(N // BLK,),
        in_specs=[part_spec, part_spec, part_spec, part_spec],
        out_specs=part_spec,
        out_shape=jax.ShapeDtypeStruct((N, H), jnp.float32),
    )(p00, p01, p10, p11)


def _split_parts(parts):
    return (parts[0:N], parts[NPAD:NPAD + N],
            parts[2 * NPAD:2 * NPAD + N], parts[3 * NPAD:3 * NPAD + N])


def kernel(x, edge_index_1, edge_index_2, W1_1, W1_2, W2_1, W2_2):
    src_idx, dst_idx = _prep_edges(edge_index_1, edge_index_2)

    table1 = _tc_project_l1(x, jnp.stack([W1_1, W1_2]))
    parts1 = _sc_propagate(table1, src_idx, dst_idx)
    table2 = _tc_combine_project(*_split_parts(parts1),
                                 jnp.stack([W2_1, W2_2]))
    parts2 = _sc_propagate(table2, src_idx, dst_idx)
    return _tc_combine(*_split_parts(parts2))


# speedup vs baseline: 4.8868x; 1.2124x over previous
import functools

import jax
import jax.numpy as jnp
from jax import lax
from jax.experimental import pallas as pl
from jax.experimental.pallas import tpu as pltpu
from jax.experimental.pallas import tpu_sc as plsc

N = 10000
E = 160000
D_IN = 128
H = 64

NC = 2
NS = 16
CH = 128
E_PAD = 163840
NCH = E_PAD // (NS * CH)
NPAD = 10240
ZCH = NPAD // NS // CH
TROWS = 2 * N

BLK = 400


def _prep_edges(edge_index_1, edge_index_2):
    pad = E_PAD - E
    zpad = jnp.zeros((pad,), jnp.int32)
    dpad = jnp.full((pad,), N, jnp.int32)
    src1 = jnp.concatenate([edge_index_1[0], zpad])
    dst1 = jnp.concatenate([edge_index_1[1], dpad])
    src2 = jnp.concatenate([edge_index_2[0] + N, zpad])
    dst2 = jnp.concatenate([edge_index_2[1], dpad])
    src_idx = jnp.concatenate(
        [src1.reshape(NS, NCH, CH), src2.reshape(NS, NCH, CH)])
    dst_idx = jnp.concatenate(
        [dst1.reshape(NS, NCH, CH), dst2.reshape(NS, NCH, CH)])
    return src_idx, dst_idx


def _sc_propagate(table, src_idx, dst_idx):
    mesh = plsc.VectorSubcoreMesh(core_axis_name="c", subcore_axis_name="s")

    @functools.partial(
        pl.kernel,
        out_type=jax.ShapeDtypeStruct((NC * NPAD, H), jnp.float32),
        mesh=mesh,
        scratch_types=[
            pltpu.VMEM((NCH, CH), jnp.int32),
            pltpu.VMEM((NCH, CH), jnp.int32),
            [pltpu.VMEM((CH, H), jnp.float32)] * 4,
            pltpu.VMEM_SHARED((NPAD, H), jnp.float32),
            [pltpu.SemaphoreType.DMA] * 4,
            [pltpu.SemaphoreType.DMA] * 4,
        ],
        compiler_params=pltpu.CompilerParams(use_tc_tiling_on_sc=False),
    )
    def prop(table_hbm, src_hbm, dst_hbm, out_hbm, src_v, dst_v, rows,
             acc_sh, gsem, ssem):
        c = lax.axis_index("c")
        s = lax.axis_index("s")
        w = c * NS + s

        def zrow(r, carry):
            for k in range(H // 16):
                rows[0][r, pl.ds(k * 16, 16)] = jnp.zeros((16,), jnp.float32)
            return carry
        lax.fori_loop(0, CH, zrow, 0)
        zbase = s * (NPAD // NS)

        def zcp(i, carry):
            pltpu.sync_copy(rows[0], acc_sh.at[pl.ds(zbase + i * CH, CH)])
            return carry
        lax.fori_loop(0, ZCH, zcp, 0)

        pltpu.sync_copy(src_hbm.at[w], src_v)
        pltpu.sync_copy(dst_hbm.at[w], dst_v)
        plsc.subcore_barrier()

        def gissue(j, k):
            pltpu.async_copy(table_hbm.at[src_v.at[j]], rows[k], gsem[k])

        def gwait(j, k):
            pltpu.make_async_copy(
                table_hbm.at[src_v.at[j]], rows[k], gsem[k]).wait()

        def sissue(j, k):
            pltpu.async_copy(rows[k], acc_sh.at[dst_v.at[j]], ssem[k],
                             add=True)

        def swait(j, k):
            pltpu.make_async_copy(
                rows[k], acc_sh.at[dst_v.at[j]], ssem[k]).wait()

        gissue(0, 0)
        gissue(1, 1)

        def step4(i, carry):
            for kk in range(4):
                j = 4 * i + kk
                m = (kk + 2) % 4
                gwait(j, kk)
                sissue(j, kk)

                @pl.when(j + 2 < NCH)
                def _():
                    @pl.when(j >= 2)
                    def _():
                        swait(j - 2, m)
                    gissue(j + 2, m)
            return carry
        lax.fori_loop(0, NCH // 4, step4, 0)
        for t in range(4):
            swait(NCH - 4 + t, t)
        plsc.subcore_barrier()

        pltpu.sync_copy(
            acc_sh.at[pl.ds(zbase, NPAD // NS)],
            out_hbm.at[pl.ds(c * NPAD + zbase, NPAD // NS)])

    return prop(table, src_idx, dst_idx)


def _tc_project_l1(x, w_stack):
    def body(x_ref, w_ref, o_ref):
        o_ref[0] = jnp.dot(x_ref[...], w_ref[0],
                           preferred_element_type=jnp.float32)

    out = pl.pallas_call(
        body,
        grid=(N // BLK, 2),
        in_specs=[
            pl.BlockSpec((BLK, D_IN), lambda i, j: (i, 0)),
            pl.BlockSpec((1, D_IN, H), lambda i, j: (j, 0, 0)),
        ],
        out_specs=pl.BlockSpec((1, BLK, H), lambda i, j: (j, i, 0)),
        out_shape=jax.ShapeDtypeStruct((2, N, H), jnp.float32),
    )(x, w_stack)
    return out.reshape(TROWS, H)


def _tc_combine_project(p0, p1, w_stack):
    def body(a_ref, b_ref, w_ref, o_ref):
        h = jnp.maximum(a_ref[...] + b_ref[...], 0.0)
        o_ref[0] = jnp.dot(h, w_ref[0], preferred_element_type=jnp.float32)

    part_spec = pl.BlockSpec((BLK, H), lambda i, j: (i, 0))
    out = pl.pallas_call(
        body,
        grid=(N // BLK, 2),
        in_specs=[part_spec, part_spec,
                  pl.BlockSpec((1, H, H), lambda i, j: (j, 0, 0))],
        out_specs=pl.BlockSpec((1, BLK, H), lambda i, j: (j, i, 0)),
        out_shape=jax.ShapeDtypeStruct((2, N, H), jnp.float32),
    )(p0, p1, w_stack)
    return out.reshape(TROWS, H)


def _tc_combine(p0, p1):
    def body(a_ref, b_ref, o_ref):
        o_ref[...] = jnp.maximum(a_ref[...] + b_ref[...], 0.0)

    part_spec = pl.BlockSpec((BLK, H), lambda i: (i, 0))
    return pl.pallas_call(
        body,
        grid=(N // BLK,),
        in_specs=[part_spec, part_spec],
        out_specs=part_spec,
        out_shape=jax.ShapeDtypeStruct((N, H), jnp.float32),
    )(p0, p1)


def _split_parts(parts):
    return parts[0:N], parts[NPAD:NPAD + N]


def kernel(x, edge_index_1, edge_index_2, W1_1, W1_2, W2_1, W2_2):
    src_idx, dst_idx = _prep_edges(edge_index_1, edge_index_2)

    table1 = _tc_project_l1(x, jnp.stack([W1_1, W1_2]))
    parts1 = _sc_propagate(table1, src_idx, dst_idx)
    table2 = _tc_combine_project(*_split_parts(parts1),
                                 jnp.stack([W2_1, W2_2]))
    parts2 = _sc_propagate(table2, src_idx, dst_idx)
    return _tc_combine(*_split_parts(parts2))


# speedup vs baseline: 9.0732x; 1.8567x over previous
import functools

import jax
import jax.numpy as jnp
from jax import lax
from jax.experimental import pallas as pl
from jax.experimental.pallas import tpu as pltpu
from jax.experimental.pallas import tpu_sc as plsc

N = 10000
E = 160000
D_IN = 128
H = 64

NC = 2
NS = 16
CH = 128
E_PAD = 163840
NCH = E_PAD // (NS * CH)
HCH = NCH // 2
NPAD = 10240
ZCH = NPAD // NS // CH
TST = N // NS
TROWS = 2 * N

BLK = 400


def _prep_edges(edge_index_1, edge_index_2):
    pad = E_PAD - E
    zpad = jnp.zeros((pad,), jnp.int32)
    dpad = jnp.full((pad,), N, jnp.int32)
    src1 = jnp.concatenate([edge_index_1[0], zpad])
    dst1 = jnp.concatenate([edge_index_1[1], dpad])
    src2 = jnp.concatenate([edge_index_2[0], zpad])
    dst2 = jnp.concatenate([edge_index_2[1], dpad])
    src_idx = jnp.concatenate(
        [src1.reshape(NS, NCH, CH), src2.reshape(NS, NCH, CH)])
    dst_idx = jnp.concatenate(
        [dst1.reshape(NS, NCH, CH), dst2.reshape(NS, NCH, CH)])
    return src_idx, dst_idx


def _sc_propagate(table, src_idx, dst_idx):
    mesh = plsc.VectorSubcoreMesh(core_axis_name="c", subcore_axis_name="s")

    @functools.partial(
        pl.kernel,
        out_type=jax.ShapeDtypeStruct((NC * NPAD, H), jnp.float32),
        mesh=mesh,
        scratch_types=[
            pltpu.VMEM((HCH, CH), jnp.int32),
            pltpu.VMEM((HCH, CH), jnp.int32),
            [pltpu.VMEM((CH, H), jnp.float32)] * 4,
            pltpu.VMEM_SHARED((N, H), jnp.float32),
            pltpu.VMEM_SHARED((NPAD, H), jnp.float32),
            [pltpu.SemaphoreType.DMA] * 4,
            [pltpu.SemaphoreType.DMA] * 4,
        ],
        compiler_params=pltpu.CompilerParams(use_tc_tiling_on_sc=False),
    )
    def prop(table_hbm, src_hbm, dst_hbm, out_hbm, src_v, dst_v, rows,
             tab_sh, acc_sh, gsem, ssem):
        c = lax.axis_index("c")
        s = lax.axis_index("s")
        w = c * NS + s

        def zrow(r, carry):
            for k in range(H // 16):
                rows[0][r, pl.ds(k * 16, 16)] = jnp.zeros((16,), jnp.float32)
            return carry
        lax.fori_loop(0, CH, zrow, 0)
        zbase = s * (NPAD // NS)

        def zcp(i, carry):
            pltpu.sync_copy(rows[0], acc_sh.at[pl.ds(zbase + i * CH, CH)])
            return carry
        lax.fori_loop(0, ZCH, zcp, 0)

        pltpu.sync_copy(
            table_hbm.at[pl.ds(c * N + s * TST, TST)],
            tab_sh.at[pl.ds(s * TST, TST)])

        def gissue(j, k):
            pltpu.async_copy(tab_sh.at[src_v.at[j]], rows[k], gsem[k])

        def gwait(j, k):
            pltpu.make_async_copy(
                tab_sh.at[src_v.at[j]], rows[k], gsem[k]).wait()

        def sissue(j, k):
            pltpu.async_copy(rows[k], acc_sh.at[dst_v.at[j]], ssem[k],
                             add=True)

        def swait(j, k):
            pltpu.make_async_copy(
                rows[k], acc_sh.at[dst_v.at[j]], ssem[k]).wait()

        def run_half(h):
            pltpu.sync_copy(src_hbm.at[w, pl.ds(h * HCH, HCH)], src_v)
            pltpu.sync_copy(dst_hbm.at[w, pl.ds(h * HCH, HCH)], dst_v)
            if h == 0:
                plsc.subcore_barrier()
            gissue(0, 0)
            gissue(1, 1)

            def step4(i, carry):
                for kk in range(4):
                    j = 4 * i + kk
                    m = (kk + 2) % 4
                    gwait(j, kk)
                    sissue(j, kk)

                    @pl.when(j + 2 < HCH)
                    def _():
                        @pl.when(j >= 2)
                        def _():
                            swait(j - 2, m)
                        gissue(j + 2, m)
                return carry
            lax.fori_loop(0, HCH // 4, step4, 0)
            for t in range(4):
                swait(HCH - 4 + t, t)

        run_half(0)
        run_half(1)
        plsc.subcore_barrier()

        pltpu.sync_copy(
            acc_sh.at[pl.ds(zbase, NPAD // NS)],
            out_hbm.at[pl.ds(c * NPAD + zbase, NPAD // NS)])

    return prop(table, src_idx, dst_idx)


def _tc_project_l1(x, w_stack):
    def body(x_ref, w_ref, o_ref):
        o_ref[0] = jnp.dot(x_ref[...], w_ref[0],
                           preferred_element_type=jnp.float32)

    out = pl.pallas_call(
        body,
        grid=(N // BLK, 2),
        in_specs=[
            pl.BlockSpec((BLK, D_IN), lambda i, j: (i, 0)),
            pl.BlockSpec((1, D_IN, H), lambda i, j: (j, 0, 0)),
        ],
        out_specs=pl.BlockSpec((1, BLK, H), lambda i, j: (j, i, 0)),
        out_shape=jax.ShapeDtypeStruct((2, N, H), jnp.float32),
    )(x, w_stack)
    return out.reshape(TROWS, H)


def _tc_combine_project(p0, p1, w_stack):
    def body(a_ref, b_ref, w_ref, o_ref):
        h = jnp.maximum(a_ref[...] + b_ref[...], 0.0)
        o_ref[0] = jnp.dot(h, w_ref[0], preferred_element_type=jnp.float32)

    part_spec = pl.BlockSpec((BLK, H), lambda i, j: (i, 0))
    out = pl.pallas_call(
        body,
        grid=(N // BLK, 2),
        in_specs=[part_spec, part_spec,
                  pl.BlockSpec((1, H, H), lambda i, j: (j, 0, 0))],
        out_specs=pl.BlockSpec((1, BLK, H), lambda i, j: (j, i, 0)),
        out_shape=jax.ShapeDtypeStruct((2, N, H), jnp.float32),
    )(p0, p1, w_stack)
    return out.reshape(TROWS, H)


def _tc_combine(p0, p1):
    def body(a_ref, b_ref, o_ref):
        o_ref[...] = jnp.maximum(a_ref[...] + b_ref[...], 0.0)

    part_spec = pl.BlockSpec((BLK, H), lambda i: (i, 0))
    return pl.pallas_call(
        body,
        grid=(N // BLK,),
        in_specs=[part_spec, part_spec],
        out_specs=part_spec,
        out_shape=jax.ShapeDtypeStruct((N, H), jnp.float32),
    )(p0, p1)


def _split_parts(parts):
    return parts[0:N], parts[NPAD:NPAD + N]


def kernel(x, edge_index_1, edge_index_2, W1_1, W1_2, W2_1, W2_2):
    src_idx, dst_idx = _prep_edges(edge_index_1, edge_index_2)

    table1 = _tc_project_l1(x, jnp.stack([W1_1, W1_2]))
    parts1 = _sc_propagate(table1, src_idx, dst_idx)
    table2 = _tc_combine_project(*_split_parts(parts1),
                                 jnp.stack([W2_1, W2_2]))
    parts2 = _sc_propagate(table2, src_idx, dst_idx)
    return _tc_combine(*_split_parts(parts2))


# speedup vs baseline: 9.7554x; 1.0752x over previous
import functools

import jax
import jax.numpy as jnp
from jax import lax
from jax.experimental import pallas as pl
from jax.experimental.pallas import tpu as pltpu
from jax.experimental.pallas import tpu_sc as plsc

N = 10000
E = 160000
D_IN = 128
H = 64

NC = 2
NS = 16
CH = 128
E_PAD = 163840
NCH = E_PAD // (NS * CH)
HCH = NCH // 2
NPAD = 10240
ZCH = NPAD // NS // CH
TST = N // NS
TROWS = 2 * N

BLK = 400


def _prep_edges(edge_index_1, edge_index_2):
    pad = E_PAD - E
    src_idx = jnp.pad(
        jnp.stack([edge_index_1[0], edge_index_2[0]]),
        ((0, 0), (0, pad))).reshape(NC * NS, NCH, CH)
    dst_idx = jnp.pad(
        jnp.stack([edge_index_1[1], edge_index_2[1]]),
        ((0, 0), (0, pad)), constant_values=N).reshape(NC * NS, NCH, CH)
    return src_idx, dst_idx


def _sc_propagate(table, src_idx, dst_idx):
    mesh = plsc.VectorSubcoreMesh(core_axis_name="c", subcore_axis_name="s")

    @functools.partial(
        pl.kernel,
        out_type=jax.ShapeDtypeStruct((NC * NPAD, H), jnp.float32),
        mesh=mesh,
        scratch_types=[
            pltpu.VMEM((HCH, CH), jnp.int32),
            pltpu.VMEM((HCH, CH), jnp.int32),
            [pltpu.VMEM((CH, H), jnp.float32)] * 4,
            pltpu.VMEM_SHARED((N, H), jnp.float32),
            pltpu.VMEM_SHARED((NPAD, H), jnp.float32),
            [pltpu.SemaphoreType.DMA] * 4,
            [pltpu.SemaphoreType.DMA] * 4,
        ],
        compiler_params=pltpu.CompilerParams(use_tc_tiling_on_sc=False),
    )
    def prop(table_hbm, src_hbm, dst_hbm, out_hbm, src_v, dst_v, rows,
             tab_sh, acc_sh, gsem, ssem):
        c = lax.axis_index("c")
        s = lax.axis_index("s")
        w = c * NS + s

        def zrow(r, carry):
            for k in range(H // 16):
                rows[0][r, pl.ds(k * 16, 16)] = jnp.zeros((16,), jnp.float32)
            return carry
        lax.fori_loop(0, CH, zrow, 0)
        zbase = s * (NPAD // NS)

        def zcp(i, carry):
            pltpu.sync_copy(rows[0], acc_sh.at[pl.ds(zbase + i * CH, CH)])
            return carry
        lax.fori_loop(0, ZCH, zcp, 0)

        pltpu.sync_copy(
            table_hbm.at[pl.ds(c * N + s * TST, TST)],
            tab_sh.at[pl.ds(s * TST, TST)])

        def gissue(j, k):
            pltpu.async_copy(tab_sh.at[src_v.at[j]], rows[k], gsem[k])

        def gwait(j, k):
            pltpu.make_async_copy(
                tab_sh.at[src_v.at[j]], rows[k], gsem[k]).wait()

        def sissue(j, k):
            pltpu.async_copy(rows[k], acc_sh.at[dst_v.at[j]], ssem[k],
                             add=True)

        def swait(j, k):
            pltpu.make_async_copy(
                rows[k], acc_sh.at[dst_v.at[j]], ssem[k]).wait()

        def run_half(h):
            pltpu.sync_copy(src_hbm.at[w, pl.ds(h * HCH, HCH)], src_v)
            pltpu.sync_copy(dst_hbm.at[w, pl.ds(h * HCH, HCH)], dst_v)
            if h == 0:
                plsc.subcore_barrier()
            gissue(0, 0)
            gissue(1, 1)

            def step4(i, carry):
                for kk in range(4):
                    j = 4 * i + kk
                    m = (kk + 2) % 4
                    gwait(j, kk)
                    sissue(j, kk)

                    @pl.when(j + 2 < HCH)
                    def _():
                        @pl.when(j >= 2)
                        def _():
                            swait(j - 2, m)
                        gissue(j + 2, m)
                return carry
            lax.fori_loop(0, HCH // 4, step4, 0)
            for t in range(4):
                swait(HCH - 4 + t, t)

        run_half(0)
        run_half(1)
        plsc.subcore_barrier()

        pltpu.sync_copy(
            acc_sh.at[pl.ds(zbase, NPAD // NS)],
            out_hbm.at[pl.ds(c * NPAD + zbase, NPAD // NS)])

    return prop(table, src_idx, dst_idx)


def _tc_project_l1(x, w_stack):
    def body(x_ref, w_ref, o_ref):
        o_ref[0] = jnp.dot(x_ref[...], w_ref[0],
                           preferred_element_type=jnp.float32)

    out = pl.pallas_call(
        body,
        grid=(N // BLK, 2),
        in_specs=[
            pl.BlockSpec((BLK, D_IN), lambda i, j: (i, 0)),
            pl.BlockSpec((1, D_IN, H), lambda i, j: (j, 0, 0)),
        ],
        out_specs=pl.BlockSpec((1, BLK, H), lambda i, j: (j, i, 0)),
        out_shape=jax.ShapeDtypeStruct((2, N, H), jnp.float32),
    )(x, w_stack)
    return out.reshape(TROWS, H)


def _tc_combine_project(parts, w_stack):
    def body(a_ref, b_ref, w_ref, o_ref):
        h = jnp.maximum(a_ref[0] + b_ref[0], 0.0)
        o_ref[0] = jnp.dot(h, w_ref[0], preferred_element_type=jnp.float32)

    p0_spec = pl.BlockSpec((1, BLK, H), lambda i, j: (0, i, 0))
    p1_spec = pl.BlockSpec((1, BLK, H), lambda i, j: (1, i, 0))
    out = pl.pallas_call(
        body,
        grid=(N // BLK, 2),
        in_specs=[p0_spec, p1_spec,
                  pl.BlockSpec((1, H, H), lambda i, j: (j, 0, 0))],
        out_specs=pl.BlockSpec((1, BLK, H), lambda i, j: (j, i, 0)),
        out_shape=jax.ShapeDtypeStruct((2, N, H), jnp.float32),
    )(parts, parts, w_stack)
    return out.reshape(TROWS, H)


def _tc_combine(parts):
    def body(a_ref, b_ref, o_ref):
        o_ref[...] = jnp.maximum(a_ref[0] + b_ref[0], 0.0)

    p0_spec = pl.BlockSpec((1, BLK, H), lambda i: (0, i, 0))
    p1_spec = pl.BlockSpec((1, BLK, H), lambda i: (1, i, 0))
    return pl.pallas_call(
        body,
        grid=(N // BLK,),
        in_specs=[p0_spec, p1_spec],
        out_specs=pl.BlockSpec((BLK, H), lambda i: (i, 0)),
        out_shape=jax.ShapeDtypeStruct((N, H), jnp.float32),
    )(parts, parts)


def kernel(x, edge_index_1, edge_index_2, W1_1, W1_2, W2_1, W2_2):
    src_idx, dst_idx = _prep_edges(edge_index_1, edge_index_2)

    table1 = _tc_project_l1(x, jnp.stack([W1_1, W1_2]))
    parts1 = _sc_propagate(table1, src_idx, dst_idx).reshape(NC, NPAD, H)
    table2 = _tc_combine_project(parts1, jnp.stack([W2_1, W2_2]))
    parts2 = _sc_propagate(table2, src_idx, dst_idx).reshape(NC, NPAD, H)
    return _tc_combine(parts2)


# speedup vs baseline: 9.8008x; 1.0047x over previous
import functools

import jax
import jax.numpy as jnp
from jax import lax
from jax.experimental import pallas as pl
from jax.experimental.pallas import tpu as pltpu
from jax.experimental.pallas import tpu_sc as plsc

N = 10000
E = 160000
D_IN = 128
H = 64

NC = 2
NS = 16
CH = 128
E_PAD = 163840
NCH = E_PAD // (NS * CH)
HCH = NCH // 2
NPAD = 10240
ZCH = NPAD // NS // CH
TST = N // NS
TROWS = 2 * N

BLK = 400


def _prep_edges(edge_index_1, edge_index_2):
    pad = E_PAD - E
    src_idx = jnp.pad(
        jnp.stack([edge_index_1[0], edge_index_2[0]]),
        ((0, 0), (0, pad))).reshape(NC * NS, NCH, CH)
    dst_idx = jnp.pad(
        jnp.stack([edge_index_1[1], edge_index_2[1]]),
        ((0, 0), (0, pad)), constant_values=N).reshape(NC * NS, NCH, CH)
    return src_idx, dst_idx


def _sc_propagate(table, src_idx, dst_idx, zstripe):
    mesh = plsc.VectorSubcoreMesh(core_axis_name="c", subcore_axis_name="s")

    @functools.partial(
        pl.kernel,
        out_type=jax.ShapeDtypeStruct((NC * NPAD, H), jnp.float32),
        mesh=mesh,
        scratch_types=[
            pltpu.VMEM((HCH, CH), jnp.int32),
            pltpu.VMEM((HCH, CH), jnp.int32),
            [pltpu.VMEM((CH, H), jnp.float32)] * 4,
            pltpu.VMEM_SHARED((N, H), jnp.float32),
            pltpu.VMEM_SHARED((NPAD, H), jnp.float32),
            [pltpu.SemaphoreType.DMA] * 4,
            [pltpu.SemaphoreType.DMA] * 4,
        ],
        compiler_params=pltpu.CompilerParams(use_tc_tiling_on_sc=False),
    )
    def prop(table_hbm, src_hbm, dst_hbm, z_hbm, out_hbm, src_v, dst_v,
             rows, tab_sh, acc_sh, gsem, ssem):
        c = lax.axis_index("c")
        s = lax.axis_index("s")
        w = c * NS + s
        zbase = s * (NPAD // NS)

        cz = pltpu.async_copy(z_hbm, acc_sh.at[pl.ds(zbase, NPAD // NS)],
                              ssem[0])
        ct = pltpu.async_copy(table_hbm.at[pl.ds(c * N + s * TST, TST)],
                              tab_sh.at[pl.ds(s * TST, TST)], ssem[1])

        def gissue(j, k):
            pltpu.async_copy(tab_sh.at[src_v.at[j]], rows[k], gsem[k])

        def gwait(j, k):
            pltpu.make_async_copy(
                tab_sh.at[src_v.at[j]], rows[k], gsem[k]).wait()

        def sissue(j, k):
            pltpu.async_copy(rows[k], acc_sh.at[dst_v.at[j]], ssem[k],
                             add=True)

        def swait(j, k):
            pltpu.make_async_copy(
                rows[k], acc_sh.at[dst_v.at[j]], ssem[k]).wait()

        def run_half(h):
            if h == 0:
                ci1 = pltpu.async_copy(
                    src_hbm.at[w, pl.ds(0, HCH)], src_v, ssem[2])
                ci2 = pltpu.async_copy(
                    dst_hbm.at[w, pl.ds(0, HCH)], dst_v, ssem[3])
                cz.wait()
                ct.wait()
                ci1.wait()
                ci2.wait()
                plsc.subcore_barrier()
            else:
                pltpu.sync_copy(src_hbm.at[w, pl.ds(h * HCH, HCH)], src_v)
                pltpu.sync_copy(dst_hbm.at[w, pl.ds(h * HCH, HCH)], dst_v)
            gissue(0, 0)
            gissue(1, 1)

            def step4(i, carry):
                for kk in range(4):
                    j = 4 * i + kk
                    m = (kk + 2) % 4
                    gwait(j, kk)
                    sissue(j, kk)

                    @pl.when(j + 2 < HCH)
                    def _():
                        @pl.when(j >= 2)
                        def _():
                            swait(j - 2, m)
                        gissue(j + 2, m)
                return carry
            lax.fori_loop(0, HCH // 4, step4, 0)
            for t in range(4):
                swait(HCH - 4 + t, t)

        run_half(0)
        run_half(1)
        plsc.subcore_barrier()

        pltpu.sync_copy(
            acc_sh.at[pl.ds(zbase, NPAD // NS)],
            out_hbm.at[pl.ds(c * NPAD + zbase, NPAD // NS)])

    return prop(table, src_idx, dst_idx, zstripe)


def _tc_project_l1(x, w_stack):
    def body(x_ref, w_ref, o_ref):
        o_ref[0] = jnp.dot(x_ref[...], w_ref[0],
                           preferred_element_type=jnp.float32)

    out = pl.pallas_call(
        body,
        grid=(N // BLK, 2),
        in_specs=[
            pl.BlockSpec((BLK, D_IN), lambda i, j: (i, 0)),
            pl.BlockSpec((1, D_IN, H), lambda i, j: (j, 0, 0)),
        ],
        out_specs=pl.BlockSpec((1, BLK, H), lambda i, j: (j, i, 0)),
        out_shape=jax.ShapeDtypeStruct((2, N, H), jnp.float32),
    )(x, w_stack)
    return out.reshape(TROWS, H)


def _tc_combine_project(parts, w_stack):
    def body(a_ref, b_ref, w_ref, o_ref):
        h = jnp.maximum(a_ref[0] + b_ref[0], 0.0)
        o_ref[0] = jnp.dot(h, w_ref[0], preferred_element_type=jnp.float32)

    p0_spec = pl.BlockSpec((1, BLK, H), lambda i, j: (0, i, 0))
    p1_spec = pl.BlockSpec((1, BLK, H), lambda i, j: (1, i, 0))
    out = pl.pallas_call(
        body,
        grid=(N // BLK, 2),
        in_specs=[p0_spec, p1_spec,
                  pl.BlockSpec((1, H, H), lambda i, j: (j, 0, 0))],
        out_specs=pl.BlockSpec((1, BLK, H), lambda i, j: (j, i, 0)),
        out_shape=jax.ShapeDtypeStruct((2, N, H), jnp.float32),
    )(parts, parts, w_stack)
    return out.reshape(TROWS, H)


def _tc_combine(parts):
    def body(a_ref, b_ref, o_ref):
        o_ref[...] = jnp.maximum(a_ref[0] + b_ref[0], 0.0)

    p0_spec = pl.BlockSpec((1, BLK, H), lambda i: (0, i, 0))
    p1_spec = pl.BlockSpec((1, BLK, H), lambda i: (1, i, 0))
    return pl.pallas_call(
        body,
        grid=(N // BLK,),
        in_specs=[p0_spec, p1_spec],
        out_specs=pl.BlockSpec((BLK, H), lambda i: (i, 0)),
        out_shape=jax.ShapeDtypeStruct((N, H), jnp.float32),
    )(parts, parts)


def kernel(x, edge_index_1, edge_index_2, W1_1, W1_2, W2_1, W2_2):
    src_idx, dst_idx = _prep_edges(edge_index_1, edge_index_2)
    zstripe = jnp.zeros((NPAD // NS, H), jnp.float32)

    table1 = _tc_project_l1(x, jnp.stack([W1_1, W1_2]))
    parts1 = _sc_propagate(
        table1, src_idx, dst_idx, zstripe).reshape(NC, NPAD, H)
    table2 = _tc_combine_project(parts1, jnp.stack([W2_1, W2_2]))
    parts2 = _sc_propagate(
        table2, src_idx, dst_idx, zstripe).reshape(NC, NPAD, H)
    return _tc_combine(parts2)


# speedup vs baseline: 11.8416x; 1.2082x over previous
import functools

import jax
import jax.numpy as jnp
from jax import lax
from jax.experimental import pallas as pl
from jax.experimental.pallas import tpu as pltpu
from jax.experimental.pallas import tpu_sc as plsc

N = 10000
E = 160000
D_IN = 128
H = 64

NC = 2
NS = 16
CH = 128
E_PAD = 163840
NCH = E_PAD // (NS * CH)
HCH = NCH // 2
NPAD = 10240
ZCH = NPAD // NS // CH
TST = N // NS
TROWS = 2 * N

BLK = 2000


def _prep_edges(edge_index_1, edge_index_2):
    pad = E_PAD - E
    src_idx = jnp.pad(
        jnp.stack([edge_index_1[0], edge_index_2[0]]),
        ((0, 0), (0, pad))).reshape(NC * NS, NCH, CH)
    dst_idx = jnp.pad(
        jnp.stack([edge_index_1[1], edge_index_2[1]]),
        ((0, 0), (0, pad)), constant_values=N).reshape(NC * NS, NCH, CH)
    return src_idx, dst_idx


def _sc_propagate(table, src_idx, dst_idx, zstripe):
    mesh = plsc.VectorSubcoreMesh(core_axis_name="c", subcore_axis_name="s")

    @functools.partial(
        pl.kernel,
        out_type=jax.ShapeDtypeStruct((NC * NPAD, H), jnp.float32),
        mesh=mesh,
        scratch_types=[
            pltpu.VMEM((HCH, CH), jnp.int32),
            pltpu.VMEM((HCH, CH), jnp.int32),
            [pltpu.VMEM((CH, H), jnp.float32)] * 4,
            pltpu.VMEM_SHARED((N, H), jnp.float32),
            pltpu.VMEM_SHARED((NPAD, H), jnp.float32),
            [pltpu.SemaphoreType.DMA] * 4,
            [pltpu.SemaphoreType.DMA] * 4,
        ],
        compiler_params=pltpu.CompilerParams(use_tc_tiling_on_sc=False),
    )
    def prop(table_hbm, src_hbm, dst_hbm, z_hbm, out_hbm, src_v, dst_v,
             rows, tab_sh, acc_sh, gsem, ssem):
        c = lax.axis_index("c")
        s = lax.axis_index("s")
        w = c * NS + s
        zbase = s * (NPAD // NS)

        cz = pltpu.async_copy(z_hbm, acc_sh.at[pl.ds(zbase, NPAD // NS)],
                              ssem[0])
        ct = pltpu.async_copy(table_hbm.at[pl.ds(c * N + s * TST, TST)],
                              tab_sh.at[pl.ds(s * TST, TST)], ssem[1])

        def gissue(j, k):
            pltpu.async_copy(tab_sh.at[src_v.at[j]], rows[k], gsem[k])

        def gwait(j, k):
            pltpu.make_async_copy(
                tab_sh.at[src_v.at[j]], rows[k], gsem[k]).wait()

        def sissue(j, k):
            pltpu.async_copy(rows[k], acc_sh.at[dst_v.at[j]], ssem[k],
                             add=True)

        def swait(j, k):
            pltpu.make_async_copy(
                rows[k], acc_sh.at[dst_v.at[j]], ssem[k]).wait()

        def run_half(h):
            if h == 0:
                ci1 = pltpu.async_copy(
                    src_hbm.at[w, pl.ds(0, HCH)], src_v, ssem[2])
                ci2 = pltpu.async_copy(
                    dst_hbm.at[w, pl.ds(0, HCH)], dst_v, ssem[3])
                cz.wait()
                ct.wait()
                ci1.wait()
                ci2.wait()
                plsc.subcore_barrier()
            else:
                pltpu.sync_copy(src_hbm.at[w, pl.ds(h * HCH, HCH)], src_v)
                pltpu.sync_copy(dst_hbm.at[w, pl.ds(h * HCH, HCH)], dst_v)
            gissue(0, 0)
            gissue(1, 1)

            def step4(i, carry):
                for kk in range(4):
                    j = 4 * i + kk
                    m = (kk + 2) % 4
                    gwait(j, kk)
                    sissue(j, kk)

                    @pl.when(j + 2 < HCH)
                    def _():
                        @pl.when(j >= 2)
                        def _():
                            swait(j - 2, m)
                        gissue(j + 2, m)
                return carry
            lax.fori_loop(0, HCH // 4, step4, 0)
            for t in range(4):
                swait(HCH - 4 + t, t)

        run_half(0)
        run_half(1)
        plsc.subcore_barrier()

        pltpu.sync_copy(
            acc_sh.at[pl.ds(zbase, NPAD // NS)],
            out_hbm.at[pl.ds(c * NPAD + zbase, NPAD // NS)])

    return prop(table, src_idx, dst_idx, zstripe)


def _tc_project_l1(x, w_stack):
    def body(x_ref, w_ref, o_ref):
        o_ref[0] = jnp.dot(x_ref[...], w_ref[0],
                           preferred_element_type=jnp.float32)

    out = pl.pallas_call(
        body,
        grid=(N // BLK, 2),
        in_specs=[
            pl.BlockSpec((BLK, D_IN), lambda i, j: (i, 0)),
            pl.BlockSpec((1, D_IN, H), lambda i, j: (j, 0, 0)),
        ],
        out_specs=pl.BlockSpec((1, BLK, H), lambda i, j: (j, i, 0)),
        out_shape=jax.ShapeDtypeStruct((2, N, H), jnp.float32),
    )(x, w_stack)
    return out.reshape(TROWS, H)


def _tc_combine_project(parts, w_stack):
    def body(a_ref, b_ref, w_ref, o_ref):
        h = jnp.maximum(a_ref[0] + b_ref[0], 0.0)
        o_ref[0] = jnp.dot(h, w_ref[0], preferred_element_type=jnp.float32)

    p0_spec = pl.BlockSpec((1, BLK, H), lambda i, j: (0, i, 0))
    p1_spec = pl.BlockSpec((1, BLK, H), lambda i, j: (1, i, 0))
    out = pl.pallas_call(
        body,
        grid=(N // BLK, 2),
        in_specs=[p0_spec, p1_spec,
                  pl.BlockSpec((1, H, H), lambda i, j: (j, 0, 0))],
        out_specs=pl.BlockSpec((1, BLK, H), lambda i, j: (j, i, 0)),
        out_shape=jax.ShapeDtypeStruct((2, N, H), jnp.float32),
    )(parts, parts, w_stack)
    return out.reshape(TROWS, H)


def _tc_combine(parts):
    def body(a_ref, b_ref, o_ref):
        o_ref[...] = jnp.maximum(a_ref[0] + b_ref[0], 0.0)

    p0_spec = pl.BlockSpec((1, BLK, H), lambda i: (0, i, 0))
    p1_spec = pl.BlockSpec((1, BLK, H), lambda i: (1, i, 0))
    return pl.pallas_call(
        body,
        grid=(N // BLK,),
        in_specs=[p0_spec, p1_spec],
        out_specs=pl.BlockSpec((BLK, H), lambda i: (i, 0)),
        out_shape=jax.ShapeDtypeStruct((N, H), jnp.float32),
    )(parts, parts)


def kernel(x, edge_index_1, edge_index_2, W1_1, W1_2, W2_1, W2_2):
    src_idx, dst_idx = _prep_edges(edge_index_1, edge_index_2)
    zstripe = jnp.zeros((NPAD // NS, H), jnp.float32)

    table1 = _tc_project_l1(x, jnp.stack([W1_1, W1_2]))
    parts1 = _sc_propagate(
        table1, src_idx, dst_idx, zstripe).reshape(NC, NPAD, H)
    table2 = _tc_combine_project(parts1, jnp.stack([W2_1, W2_2]))
    parts2 = _sc_propagate(
        table2, src_idx, dst_idx, zstripe).reshape(NC, NPAD, H)
    return _tc_combine(parts2)


# speedup vs baseline: 11.8474x; 1.0005x over previous
import functools

import jax
import jax.numpy as jnp
from jax import lax
from jax.experimental import pallas as pl
from jax.experimental.pallas import tpu as pltpu
from jax.experimental.pallas import tpu_sc as plsc

N = 10000
E = 160000
D_IN = 128
H = 64

NC = 2
NS = 16
CH = 64
E_PAD = 163840
NCH = E_PAD // (NS * CH)
HCH = NCH // 2
RING = 8
NPAD = 10240
ZCH = NPAD // NS // CH
TST = N // NS
TROWS = 2 * N

BLK = 2000


def _prep_edges(edge_index_1, edge_index_2):
    pad = E_PAD - E
    src_idx = jnp.pad(
        jnp.stack([edge_index_1[0], edge_index_2[0]]),
        ((0, 0), (0, pad))).reshape(NC * NS, NCH, CH)
    dst_idx = jnp.pad(
        jnp.stack([edge_index_1[1], edge_index_2[1]]),
        ((0, 0), (0, pad)), constant_values=N).reshape(NC * NS, NCH, CH)
    return src_idx, dst_idx


def _sc_propagate(table, src_idx, dst_idx, zstripe):
    mesh = plsc.VectorSubcoreMesh(core_axis_name="c", subcore_axis_name="s")

    @functools.partial(
        pl.kernel,
        out_type=jax.ShapeDtypeStruct((NC * NPAD, H), jnp.float32),
        mesh=mesh,
        scratch_types=[
            pltpu.VMEM((HCH, CH), jnp.int32),
            pltpu.VMEM((HCH, CH), jnp.int32),
            [pltpu.VMEM((CH, H), jnp.float32)] * RING,
            pltpu.VMEM_SHARED((N, H), jnp.float32),
            pltpu.VMEM_SHARED((NPAD, H), jnp.float32),
            [pltpu.SemaphoreType.DMA] * RING,
            [pltpu.SemaphoreType.DMA] * RING,
        ],
        compiler_params=pltpu.CompilerParams(use_tc_tiling_on_sc=False),
    )
    def prop(table_hbm, src_hbm, dst_hbm, z_hbm, out_hbm, src_v, dst_v,
             rows, tab_sh, acc_sh, gsem, ssem):
        c = lax.axis_index("c")
        s = lax.axis_index("s")
        w = c * NS + s
        zbase = s * (NPAD // NS)

        cz = pltpu.async_copy(z_hbm, acc_sh.at[pl.ds(zbase, NPAD // NS)],
                              ssem[0])
        ct = pltpu.async_copy(table_hbm.at[pl.ds(c * N + s * TST, TST)],
                              tab_sh.at[pl.ds(s * TST, TST)], ssem[1])

        def gissue(j, k):
            pltpu.async_copy(tab_sh.at[src_v.at[j]], rows[k], gsem[k])

        def gwait(j, k):
            pltpu.make_async_copy(
                tab_sh.at[src_v.at[j]], rows[k], gsem[k]).wait()

        def sissue(j, k):
            pltpu.async_copy(rows[k], acc_sh.at[dst_v.at[j]], ssem[k],
                             add=True)

        def swait(j, k):
            pltpu.make_async_copy(
                rows[k], acc_sh.at[dst_v.at[j]], ssem[k]).wait()

        def run_half(h):
            if h == 0:
                ci1 = pltpu.async_copy(
                    src_hbm.at[w, pl.ds(0, HCH)], src_v, ssem[2])
                ci2 = pltpu.async_copy(
                    dst_hbm.at[w, pl.ds(0, HCH)], dst_v, ssem[3])
                cz.wait()
                ct.wait()
                ci1.wait()
                ci2.wait()
                plsc.subcore_barrier()
            else:
                pltpu.sync_copy(src_hbm.at[w, pl.ds(h * HCH, HCH)], src_v)
                pltpu.sync_copy(dst_hbm.at[w, pl.ds(h * HCH, HCH)], dst_v)
            lead = RING // 2
            for t in range(lead):
                gissue(t, t)

            def stepn(i, carry):
                for kk in range(RING):
                    j = RING * i + kk
                    m = (kk + lead) % RING
                    gwait(j, kk)
                    sissue(j, kk)

                    @pl.when(j + lead < HCH)
                    def _():
                        @pl.when(j >= lead)
                        def _():
                            swait(j - lead, m)
                        gissue(j + lead, m)
                return carry
            lax.fori_loop(0, HCH // RING, stepn, 0)
            for t in range(RING):
                swait(HCH - RING + t, t)

        run_half(0)
        run_half(1)
        plsc.subcore_barrier()

        pltpu.sync_copy(
            acc_sh.at[pl.ds(zbase, NPAD // NS)],
            out_hbm.at[pl.ds(c * NPAD + zbase, NPAD // NS)])

    return prop(table, src_idx, dst_idx, zstripe)


def _tc_project_l1(x, w_stack):
    def body(x_ref, w_ref, o_ref):
        o_ref[0] = jnp.dot(x_ref[...], w_ref[0],
                           preferred_element_type=jnp.float32)

    out = pl.pallas_call(
        body,
        grid=(N // BLK, 2),
        in_specs=[
            pl.BlockSpec((BLK, D_IN), lambda i, j: (i, 0)),
            pl.BlockSpec((1, D_IN, H), lambda i, j: (j, 0, 0)),
        ],
        out_specs=pl.BlockSpec((1, BLK, H), lambda i, j: (j, i, 0)),
        out_shape=jax.ShapeDtypeStruct((2, N, H), jnp.float32),
    )(x, w_stack)
    return out.reshape(TROWS, H)


def _tc_combine_project(parts, w_stack):
    def body(a_ref, b_ref, w_ref, o_ref):
        h = jnp.maximum(a_ref[0] + b_ref[0], 0.0)
        o_ref[0] = jnp.dot(h, w_ref[0], preferred_element_type=jnp.float32)

    p0_spec = pl.BlockSpec((1, BLK, H), lambda i, j: (0, i, 0))
    p1_spec = pl.BlockSpec((1, BLK, H), lambda i, j: (1, i, 0))
    out = pl.pallas_call(
        body,
        grid=(N // BLK, 2),
        in_specs=[p0_spec, p1_spec,
                  pl.BlockSpec((1, H, H), lambda i, j: (j, 0, 0))],
        out_specs=pl.BlockSpec((1, BLK, H), lambda i, j: (j, i, 0)),
        out_shape=jax.ShapeDtypeStruct((2, N, H), jnp.float32),
    )(parts, parts, w_stack)
    return out.reshape(TROWS, H)


def _tc_combine(parts):
    def body(a_ref, b_ref, o_ref):
        o_ref[...] = jnp.maximum(a_ref[0] + b_ref[0], 0.0)

    p0_spec = pl.BlockSpec((1, BLK, H), lambda i: (0, i, 0))
    p1_spec = pl.BlockSpec((1, BLK, H), lambda i: (1, i, 0))
    return pl.pallas_call(
        body,
        grid=(N // BLK,),
        in_specs=[p0_spec, p1_spec],
        out_specs=pl.BlockSpec((BLK, H), lambda i: (i, 0)),
        out_shape=jax.ShapeDtypeStruct((N, H), jnp.float32),
    )(parts, parts)


def kernel(x, edge_index_1, edge_index_2, W1_1, W1_2, W2_1, W2_2):
    src_idx, dst_idx = _prep_edges(edge_index_1, edge_index_2)
    zstripe = jnp.zeros((NPAD // NS, H), jnp.float32)

    table1 = _tc_project_l1(x, jnp.stack([W1_1, W1_2]))
    parts1 = _sc_propagate(
        table1, src_idx, dst_idx, zstripe).reshape(NC, NPAD, H)
    table2 = _tc_combine_project(parts1, jnp.stack([W2_1, W2_2]))
    parts2 = _sc_propagate(
        table2, src_idx, dst_idx, zstripe).reshape(NC, NPAD, H)
    return _tc_combine(parts2)


# speedup vs baseline: 12.4568x; 1.0514x over previous
import functools

import jax
import jax.numpy as jnp
from jax import lax
from jax.experimental import pallas as pl
from jax.experimental.pallas import tpu as pltpu
from jax.experimental.pallas import tpu_sc as plsc

N = 10000
E = 160000
D_IN = 128
H = 64

NC = 2
NS = 16
CH = 64
E_PAD = 163840
NCH = E_PAD // (NS * CH)
HCH = NCH // 2
RING = 8
NPAD = 10240
ZCH = NPAD // NS // CH
TST = N // NS
TROWS = 2 * N

BLK = 10000


def _prep_edges(edge_index_1, edge_index_2):
    pad = E_PAD - E
    src_idx = jnp.pad(
        jnp.stack([edge_index_1[0], edge_index_2[0]]),
        ((0, 0), (0, pad))).reshape(NC * NS, NCH, CH)
    dst_idx = jnp.pad(
        jnp.stack([edge_index_1[1], edge_index_2[1]]),
        ((0, 0), (0, pad)), constant_values=N).reshape(NC * NS, NCH, CH)
    return src_idx, dst_idx


def _sc_propagate(table, src_idx, dst_idx, zstripe):
    mesh = plsc.VectorSubcoreMesh(core_axis_name="c", subcore_axis_name="s")

    @functools.partial(
        pl.kernel,
        out_type=jax.ShapeDtypeStruct((NC * NPAD, H), jnp.float32),
        mesh=mesh,
        scratch_types=[
            pltpu.VMEM((HCH, CH), jnp.int32),
            pltpu.VMEM((HCH, CH), jnp.int32),
            [pltpu.VMEM((CH, H), jnp.float32)] * RING,
            pltpu.VMEM_SHARED((N, H), jnp.float32),
            pltpu.VMEM_SHARED((NPAD, H), jnp.float32),
            [pltpu.SemaphoreType.DMA] * RING,
            [pltpu.SemaphoreType.DMA] * RING,
        ],
        compiler_params=pltpu.CompilerParams(use_tc_tiling_on_sc=False),
    )
    def prop(table_hbm, src_hbm, dst_hbm, z_hbm, out_hbm, src_v, dst_v,
             rows, tab_sh, acc_sh, gsem, ssem):
        c = lax.axis_index("c")
        s = lax.axis_index("s")
        w = c * NS + s
        zbase = s * (NPAD // NS)

        cz = pltpu.async_copy(z_hbm, acc_sh.at[pl.ds(zbase, NPAD // NS)],
                              ssem[0])
        ct = pltpu.async_copy(table_hbm.at[pl.ds(c * N + s * TST, TST)],
                              tab_sh.at[pl.ds(s * TST, TST)], ssem[1])

        def gissue(j, k):
            pltpu.async_copy(tab_sh.at[src_v.at[j]], rows[k], gsem[k])

        def gwait(j, k):
            pltpu.make_async_copy(
                tab_sh.at[src_v.at[j]], rows[k], gsem[k]).wait()

        def sissue(j, k):
            pltpu.async_copy(rows[k], acc_sh.at[dst_v.at[j]], ssem[k],
                             add=True)

        def swait(j, k):
            pltpu.make_async_copy(
                rows[k], acc_sh.at[dst_v.at[j]], ssem[k]).wait()

        def run_half(h):
            if h == 0:
                ci1 = pltpu.async_copy(
                    src_hbm.at[w, pl.ds(0, HCH)], src_v, ssem[2])
                ci2 = pltpu.async_copy(
                    dst_hbm.at[w, pl.ds(0, HCH)], dst_v, ssem[3])
                cz.wait()
                ct.wait()
                ci1.wait()
                ci2.wait()
                plsc.subcore_barrier()
            else:
                pltpu.sync_copy(src_hbm.at[w, pl.ds(h * HCH, HCH)], src_v)
                pltpu.sync_copy(dst_hbm.at[w, pl.ds(h * HCH, HCH)], dst_v)
            lead = RING // 2
            for t in range(lead):
                gissue(t, t)

            def stepn(i, carry):
                for kk in range(RING):
                    j = RING * i + kk
                    m = (kk + lead) % RING
                    gwait(j, kk)
                    sissue(j, kk)

                    @pl.when(j + lead < HCH)
                    def _():
                        @pl.when(j >= lead)
                        def _():
                            swait(j - lead, m)
                        gissue(j + lead, m)
                return carry
            lax.fori_loop(0, HCH // RING, stepn, 0)
            for t in range(RING):
                swait(HCH - RING + t, t)

        run_half(0)
        run_half(1)
        plsc.subcore_barrier()

        pltpu.sync_copy(
            acc_sh.at[pl.ds(zbase, NPAD // NS)],
            out_hbm.at[pl.ds(c * NPAD + zbase, NPAD // NS)])

    return prop(table, src_idx, dst_idx, zstripe)


def _tc_project_l1(x, w_stack):
    def body(x_ref, w_ref, o_ref):
        o_ref[0] = jnp.dot(x_ref[...], w_ref[0],
                           preferred_element_type=jnp.float32)

    out = pl.pallas_call(
        body,
        grid=(N // BLK, 2),
        in_specs=[
            pl.BlockSpec((BLK, D_IN), lambda i, j: (i, 0)),
            pl.BlockSpec((1, D_IN, H), lambda i, j: (j, 0, 0)),
        ],
        out_specs=pl.BlockSpec((1, BLK, H), lambda i, j: (j, i, 0)),
        out_shape=jax.ShapeDtypeStruct((2, N, H), jnp.float32),
    )(x, w_stack)
    return out.reshape(TROWS, H)


def _tc_combine_project(parts, w_stack):
    def body(a_ref, b_ref, w_ref, o_ref):
        h = jnp.maximum(a_ref[0] + b_ref[0], 0.0)
        o_ref[0] = jnp.dot(h, w_ref[0], preferred_element_type=jnp.float32)

    p0_spec = pl.BlockSpec((1, BLK, H), lambda i, j: (0, i, 0))
    p1_spec = pl.BlockSpec((1, BLK, H), lambda i, j: (1, i, 0))
    out = pl.pallas_call(
        body,
        grid=(N // BLK, 2),
        in_specs=[p0_spec, p1_spec,
                  pl.BlockSpec((1, H, H), lambda i, j: (j, 0, 0))],
        out_specs=pl.BlockSpec((1, BLK, H), lambda i, j: (j, i, 0)),
        out_shape=jax.ShapeDtypeStruct((2, N, H), jnp.float32),
    )(parts, parts, w_stack)
    return out.reshape(TROWS, H)


def _tc_combine(parts):
    def body(a_ref, b_ref, o_ref):
        o_ref[...] = jnp.maximum(a_ref[0] + b_ref[0], 0.0)

    p0_spec = pl.BlockSpec((1, BLK, H), lambda i: (0, i, 0))
    p1_spec = pl.BlockSpec((1, BLK, H), lambda i: (1, i, 0))
    return pl.pallas_call(
        body,
        grid=(N // BLK,),
        in_specs=[p0_spec, p1_spec],
        out_specs=pl.BlockSpec((BLK, H), lambda i: (i, 0)),
        out_shape=jax.ShapeDtypeStruct((N, H), jnp.float32),
    )(parts, parts)


def kernel(x, edge_index_1, edge_index_2, W1_1, W1_2, W2_1, W2_2):
    src_idx, dst_idx = _prep_edges(edge_index_1, edge_index_2)
    zstripe = jnp.zeros((NPAD // NS, H), jnp.float32)

    table1 = _tc_project_l1(x, jnp.stack([W1_1, W1_2]))
    parts1 = _sc_propagate(
        table1, src_idx, dst_idx, zstripe).reshape(NC, NPAD, H)
    table2 = _tc_combine_project(parts1, jnp.stack([W2_1, W2_2]))
    parts2 = _sc_propagate(
        table2, src_idx, dst_idx, zstripe).reshape(NC, NPAD, H)
    return _tc_combine(parts2)
